# baseline (speedup 1.0000x reference)
import functools

import jax
import jax.numpy as jnp
from jax import lax
from jax.experimental import pallas as pl
from jax.experimental.pallas import tpu as pltpu
from jax.experimental.pallas import tpu_sc as plsc

NC = 2
NS = 16
NW = NC * NS

B = 4096 * 50
D = 64
CHUNK = 128
NCHUNKS = B // CHUNK
CPW = NCHUNKS // NW
NBUF = 5


def _make_gather(num_embeddings):
    mesh = plsc.VectorSubcoreMesh(
        core_axis_name="c", subcore_axis_name="s",
        num_cores=NC, num_subcores=NS)

    @functools.partial(
        pl.kernel,
        out_type=jax.ShapeDtypeStruct((NCHUNKS, CHUNK, D), jnp.float32),
        mesh=mesh,
        scratch_types=(
            [pltpu.VMEM((CPW, CHUNK), jnp.int32)]
            + [pltpu.VMEM((CHUNK, D), jnp.float32) for _ in range(NBUF)]
            + [pltpu.SemaphoreType.DMA for _ in range(NBUF)]
        ),
        compiler_params=pltpu.CompilerParams(use_tc_tiling_on_sc=False),
    )
    def gather(idx_hbm, table_hbm, out_hbm, idx_v, *bufs_and_sems):
        rows = bufs_and_sems[:NBUF]
        sems = bufs_and_sems[NBUF:]
        wid = lax.axis_index("s") * NC + lax.axis_index("c")
        cbase = wid * CPW

        pltpu.sync_copy(idx_hbm.at[wid], idx_v)

        def fire(g, b):
            pltpu.async_copy(table_hbm.at[idx_v.at[g]], rows[b], sems[b])

        def wait(b):
            pltpu.make_async_copy(
                table_hbm.at[idx_v.at[0]], rows[b], sems[b]).wait()

        for b in range(NBUF):
            fire(b, b)

        @pl.loop(0, CPW - NBUF, step=NBUF)
        def _(outer):
            for b in range(NBUF):
                g = outer + b
                wait(b)
                pltpu.sync_copy(rows[b], out_hbm.at[cbase + g])
                fire(g + NBUF, b)

        for b in range(NBUF):
            g = (CPW - NBUF) + b
            wait(b)
            pltpu.sync_copy(rows[b], out_hbm.at[cbase + g])

    return gather


def kernel(token_ids, embedding_matrix):
    n, s = token_ids.shape
    idx = token_ids.astype(jnp.int32).reshape(NW, CPW, CHUNK)
    out = _make_gather(embedding_matrix.shape[0])(idx, embedding_matrix)
    return out.reshape(n, s, D)


# speedup vs baseline: 1.0011x; 1.0011x over previous
import functools

import jax
import jax.numpy as jnp
from jax import lax
from jax.experimental import pallas as pl
from jax.experimental.pallas import tpu as pltpu
from jax.experimental.pallas import tpu_sc as plsc

NC = 2
NS = 16
NW = NC * NS

B = 4096 * 50
D = 64
CHUNK = 640
NCHUNKS = B // CHUNK
CPW = NCHUNKS // NW
NBUF = 2


def _make_gather(num_embeddings):
    mesh = plsc.VectorSubcoreMesh(
        core_axis_name="c", subcore_axis_name="s",
        num_cores=NC, num_subcores=NS)

    @functools.partial(
        pl.kernel,
        out_type=jax.ShapeDtypeStruct((NCHUNKS, CHUNK, D), jnp.float32),
        mesh=mesh,
        scratch_types=(
            [pltpu.VMEM((CPW, CHUNK), jnp.int32)]
            + [pltpu.VMEM((CHUNK, D), jnp.float32) for _ in range(NBUF)]
            + [pltpu.SemaphoreType.DMA for _ in range(NBUF)]
        ),
        compiler_params=pltpu.CompilerParams(use_tc_tiling_on_sc=False),
    )
    def gather(idx_hbm, table_hbm, out_hbm, idx_v, *bufs_and_sems):
        rows = bufs_and_sems[:NBUF]
        sems = bufs_and_sems[NBUF:]
        wid = lax.axis_index("s") * NC + lax.axis_index("c")
        cbase = wid * CPW

        pltpu.sync_copy(idx_hbm.at[wid], idx_v)

        def fire(g, b):
            pltpu.async_copy(table_hbm.at[idx_v.at[g]], rows[b], sems[b])

        def wait(b):
            pltpu.make_async_copy(
                table_hbm.at[idx_v.at[0]], rows[b], sems[b]).wait()

        for b in range(NBUF):
            fire(b, b)

        @pl.loop(0, CPW - NBUF, step=NBUF)
        def _(outer):
            for b in range(NBUF):
                g = outer + b
                wait(b)
                pltpu.sync_copy(rows[b], out_hbm.at[cbase + g])
                fire(g + NBUF, b)

        for b in range(NBUF):
            g = (CPW - NBUF) + b
            wait(b)
            pltpu.sync_copy(rows[b], out_hbm.at[cbase + g])

    return gather


def kernel(token_ids, embedding_matrix):
    n, s = token_ids.shape
    idx = token_ids.astype(jnp.int32).reshape(NW, CPW, CHUNK)
    out = _make_gather(embedding_matrix.shape[0])(idx, embedding_matrix)
    return out.reshape(n, s, D)
